# baseline (speedup 1.0000x reference)
import jax
import jax.numpy as jnp
from jax import lax
from jax.experimental import pallas as pl
from jax.experimental.pallas import tpu as pltpu
from jax.experimental.pallas import tpu_sc as plsc

N_USERS_C = 500000
B_C = 16384
D_C = 16
NUM_CORES = 2
NUM_SUBCORES = 16
NW = NUM_CORES * NUM_SUBCORES
CHUNK = B_C // NW
GROUPS = CHUNK // D_C


def _sc_body(mf_hbm, users_hbm, items_hbm, w_hbm,
             scores_hbm, l2_hbm,
             u_idx, i_idx, u_rows, i_rows, w_v, s_out, l_out, sem):
    wid = lax.axis_index("s") * NUM_CORES + lax.axis_index("c")
    base = wid * CHUNK

    pltpu.sync_copy(users_hbm.at[pl.ds(base, CHUNK)], u_idx)
    pltpu.sync_copy(items_hbm.at[pl.ds(base, CHUNK)], i_idx)
    pltpu.sync_copy(w_hbm, w_v)

    for k in range(CHUNK // D_C):
        sl = pl.ds(k * D_C, D_C)
        i_idx[sl] = i_idx[sl] + N_USERS_C

    cp_u = pltpu.make_async_copy(mf_hbm.at[u_idx], u_rows, sem)
    cp_i = pltpu.make_async_copy(mf_hbm.at[i_idx], i_rows, sem)
    cp_u.start()
    cp_i.start()
    cp_u.wait()
    cp_i.wait()

    iota = lax.iota(jnp.int32, D_C)
    w_vec = w_v[...]
    _dn = lax.GatherDimensionNumbers(
        offset_dims=(), collapsed_slice_dims=(0,), start_index_map=(0,))

    def _bcast_lane(vec, d):
        idx = jnp.full((D_C, 1), d, jnp.int32)
        return lax.gather(vec, idx, dimension_numbers=_dn, slice_sizes=(1,),
                          mode=lax.GatherScatterMode.PROMISE_IN_BOUNDS)

    w_bcast = [_bcast_lane(w_vec, d) for d in range(D_C)]

    def group(g, carry):
        row0 = g * D_C
        rows = row0 + iota
        acc_s = jnp.zeros((D_C,), jnp.float32)
        acc_l = jnp.zeros((D_C,), jnp.float32)
        for d in range(D_C):
            col = jnp.full((D_C,), d, jnp.int32)
            u = plsc.load_gather(u_rows, [rows, col])
            it = plsc.load_gather(i_rows, [rows, col])
            s = u * it * w_bcast[d]
            acc_s = acc_s + s
            acc_l = acc_l + s * s
        s_out[pl.ds(row0, D_C)] = acc_s
        l_out[pl.ds(row0, D_C)] = acc_l
        return carry

    lax.fori_loop(0, GROUPS, group, 0)

    pltpu.sync_copy(s_out, scores_hbm.at[pl.ds(base, CHUNK)])
    pltpu.sync_copy(l_out, l2_hbm.at[pl.ds(base, CHUNK)])


def kernel(mf_table, mlp_table, W1, b1, out_w, users, items):
    users = users.astype(jnp.int32)
    items = items.astype(jnp.int32)
    w16 = out_w[0, :D_C].astype(jnp.float32)

    mesh = plsc.VectorSubcoreMesh(core_axis_name="c", subcore_axis_name="s")
    scores, l2 = pl.kernel(
        _sc_body,
        out_type=(
            jax.ShapeDtypeStruct((B_C,), jnp.float32),
            jax.ShapeDtypeStruct((B_C,), jnp.float32),
        ),
        mesh=mesh,
        compiler_params=pltpu.CompilerParams(needs_layout_passes=False,
                                             use_tc_tiling_on_sc=False),
        scratch_types=[
            pltpu.VMEM((CHUNK,), jnp.int32),
            pltpu.VMEM((CHUNK,), jnp.int32),
            pltpu.VMEM((CHUNK, D_C), jnp.float32),
            pltpu.VMEM((CHUNK, D_C), jnp.float32),
            pltpu.VMEM((D_C,), jnp.float32),
            pltpu.VMEM((CHUNK,), jnp.float32),
            pltpu.VMEM((CHUNK,), jnp.float32),
            pltpu.SemaphoreType.DMA,
        ],
    )(mf_table, users, items, w16)
    return (scores, l2)
